# Initial kernel scaffold; baseline (speedup 1.0000x reference)
import jax
import jax.numpy as jnp
from jax.experimental import pallas as pl


def kernel(x, W):
    raise NotImplementedError("write your pallas kernel here")


# speedup vs baseline: 2.5896x; 2.5896x over previous
import jax
import jax.numpy as jnp
from jax.experimental import pallas as pl
from jax.experimental.pallas import tpu as pltpu

N, D, E = 32768, 768, 2
TA = 1024
NB = N // TA
CHUNK = 256


def _route_kernel(x_ref, w_ref, out_ref, dst0_ref, dst1_ref, sel_ref, cnt):
    @pl.when(pl.program_id(0) == 0)
    def _():
        cnt[0] = 0
        cnt[1] = 0

    x = x_ref[...]
    logits = jax.lax.dot_general(x, w_ref[...], (((1,), (0,)), ((), ())),
                                 preferred_element_type=jnp.float32)
    l0 = logits[:, 0:1]
    l1 = logits[:, 1:2]
    m = jnp.maximum(l0, l1)
    e0 = jnp.exp(l0 - m)
    e1 = jnp.exp(l1 - m)
    den = e0 + e1
    g0 = e0 / den
    g1 = e1 / den
    selb = g1 > g0
    val = jnp.maximum(g0, g1)
    out_ref[...] = x * val

    ii = jax.lax.broadcasted_iota(jnp.int32, (TA, TA), 0)
    jj = jax.lax.broadcasted_iota(jnp.int32, (TA, TA), 1)
    tri = (jj < ii).astype(jnp.float32)
    m1 = selb.astype(jnp.float32)
    pos1 = jax.lax.dot_general(tri, m1, (((1,), (0,)), ((), ())),
                               preferred_element_type=jnp.float32)
    ar = jax.lax.broadcasted_iota(jnp.int32, (TA, 1), 0).astype(jnp.float32)
    pos0 = ar - pos1

    c0 = cnt[0]
    c1 = cnt[1]
    dest0 = c0 + pos0.astype(jnp.int32)
    dest1 = c1 + pos1.astype(jnp.int32)
    dst0 = jnp.where(selb, N - 1 - dest1, dest0)
    dst1 = jnp.where(selb, dest1, N - 1 - dest0)
    dst0_ref[...] = dst0.reshape(1, TA, 1)
    dst1_ref[...] = dst1.reshape(1, TA, 1)
    sel_ref[...] = selb.astype(jnp.int32).reshape(1, TA, 1)

    t1 = jnp.sum(m1).astype(jnp.int32)
    cnt[0] = c0 + (TA - t1)
    cnt[1] = c1 + t1


def _scatter_kernel(src_ref, d0_ref, d1_ref, s_ref, x0_ref, x1_ref,
                    zero_ref, sem):
    zero_ref[...] = jnp.zeros((1, D), jnp.float32)

    def issue(r, _):
        d0 = d0_ref[0, 0, r]
        d1 = d1_ref[0, 0, r]
        s = s_ref[0, 0, r]

        @pl.when(s == 0)
        def _():
            pltpu.make_async_copy(src_ref.at[r], x0_ref.at[d0], sem).start()
            pltpu.make_async_copy(zero_ref.at[0], x1_ref.at[d1], sem).start()

        @pl.when(s == 1)
        def _():
            pltpu.make_async_copy(zero_ref.at[0], x0_ref.at[d0], sem).start()
            pltpu.make_async_copy(src_ref.at[r], x1_ref.at[d1], sem).start()
        return 0

    def drain(r, _):
        pltpu.make_async_copy(zero_ref.at[0], x0_ref.at[0], sem).wait()
        return 0

    for c in range(TA // CHUNK):
        jax.lax.fori_loop(c * CHUNK, (c + 1) * CHUNK, issue, 0)
        jax.lax.fori_loop(0, 2 * CHUNK, drain, 0)


def kernel(x, W):
    out, dst0, dst1, sel = pl.pallas_call(
        _route_kernel,
        grid=(NB,),
        in_specs=[
            pl.BlockSpec((TA, D), lambda i: (i, 0)),
            pl.BlockSpec((D, E), lambda i: (0, 0)),
        ],
        out_specs=[
            pl.BlockSpec((TA, D), lambda i: (i, 0)),
            pl.BlockSpec((1, TA, 1), lambda i: (i, 0, 0)),
            pl.BlockSpec((1, TA, 1), lambda i: (i, 0, 0)),
            pl.BlockSpec((1, TA, 1), lambda i: (i, 0, 0)),
        ],
        out_shape=[
            jax.ShapeDtypeStruct((N, D), jnp.float32),
            jax.ShapeDtypeStruct((NB, TA, 1), jnp.int32),
            jax.ShapeDtypeStruct((NB, TA, 1), jnp.int32),
            jax.ShapeDtypeStruct((NB, TA, 1), jnp.int32),
        ],
        scratch_shapes=[pltpu.SMEM((2,), jnp.int32)],
        compiler_params=pltpu.CompilerParams(
            dimension_semantics=("arbitrary",)),
    )(x, W)

    dst0 = dst0.reshape(NB, 1, TA)
    dst1 = dst1.reshape(NB, 1, TA)
    sel = sel.reshape(NB, 1, TA)

    x0, x1 = pl.pallas_call(
        _scatter_kernel,
        grid=(NB,),
        in_specs=[
            pl.BlockSpec((TA, D), lambda i: (i, 0)),
            pl.BlockSpec((1, 1, TA), lambda i: (i, 0, 0),
                         memory_space=pltpu.MemorySpace.SMEM),
            pl.BlockSpec((1, 1, TA), lambda i: (i, 0, 0),
                         memory_space=pltpu.MemorySpace.SMEM),
            pl.BlockSpec((1, 1, TA), lambda i: (i, 0, 0),
                         memory_space=pltpu.MemorySpace.SMEM),
        ],
        out_specs=[
            pl.BlockSpec(memory_space=pltpu.MemorySpace.HBM),
            pl.BlockSpec(memory_space=pltpu.MemorySpace.HBM),
        ],
        out_shape=[
            jax.ShapeDtypeStruct((N, D), jnp.float32),
            jax.ShapeDtypeStruct((N, D), jnp.float32),
        ],
        scratch_shapes=[
            pltpu.VMEM((1, D), jnp.float32),
            pltpu.SemaphoreType.DMA,
        ],
        compiler_params=pltpu.CompilerParams(
            dimension_semantics=("arbitrary",)),
    )(out, dst0, dst1, sel)

    return (x0, x1, out)


# speedup vs baseline: 2.7592x; 1.0655x over previous
import jax
import jax.numpy as jnp
from jax.experimental import pallas as pl
from jax.experimental.pallas import tpu as pltpu

N, D, E = 32768, 768, 2
TA = 1024
NB = N // TA
CHUNK = 256


def _router_kernel(x_ref, w_ref, out_ref, x0_ref, x1_ref,
                   cnt, d0v, d1v, sv, d0s, d1s, ss, zero_ref, isem, sem):
    @pl.when(pl.program_id(0) == 0)
    def _():
        cnt[0] = 0
        cnt[1] = 0
        zero_ref[...] = jnp.zeros((1, D), jnp.float32)

    x = x_ref[...]
    logits = jax.lax.dot_general(x, w_ref[...], (((1,), (0,)), ((), ())),
                                 preferred_element_type=jnp.float32)
    lt = jnp.transpose(logits)
    l0 = lt[0:1, :]
    l1 = lt[1:2, :]
    m = jnp.maximum(l0, l1)
    e0 = jnp.exp(l0 - m)
    e1 = jnp.exp(l1 - m)
    den = e0 + e1
    g0 = e0 / den
    g1 = e1 / den
    selb = g1 > g0
    val = jnp.maximum(g0, g1)
    out_ref[...] = x * jnp.transpose(val)

    ii = jax.lax.broadcasted_iota(jnp.int32, (TA, TA), 0)
    jj = jax.lax.broadcasted_iota(jnp.int32, (TA, TA), 1)
    tri = (ii < jj).astype(jnp.float32)
    m1 = selb.astype(jnp.float32)
    pos1 = jax.lax.dot_general(m1, tri, (((1,), (0,)), ((), ())),
                               preferred_element_type=jnp.float32)
    ar = jax.lax.broadcasted_iota(jnp.int32, (1, TA), 1).astype(jnp.float32)
    pos0 = ar - pos1

    c0 = cnt[0]
    c1 = cnt[1]
    dest0 = c0 + pos0.astype(jnp.int32)
    dest1 = c1 + pos1.astype(jnp.int32)
    d0v[...] = jnp.where(selb, N - 1 - dest1, dest0)
    d1v[...] = jnp.where(selb, dest1, N - 1 - dest0)
    sv[...] = selb.astype(jnp.int32)

    t1 = jnp.sum(m1).astype(jnp.int32)
    cnt[0] = c0 + (TA - t1)
    cnt[1] = c1 + t1

    cp0 = pltpu.make_async_copy(d0v, d0s, isem)
    cp1 = pltpu.make_async_copy(d1v, d1s, isem)
    cp2 = pltpu.make_async_copy(sv, ss, isem)
    cp0.start(); cp1.start(); cp2.start()
    cp0.wait(); cp1.wait(); cp2.wait()

    def issue(r, _):
        d0 = d0s[0, r]
        d1 = d1s[0, r]
        s = ss[0, r]

        @pl.when(s == 0)
        def _():
            pltpu.make_async_copy(out_ref.at[r], x0_ref.at[d0], sem).start()
            pltpu.make_async_copy(zero_ref.at[0], x1_ref.at[d1], sem).start()

        @pl.when(s == 1)
        def _():
            pltpu.make_async_copy(zero_ref.at[0], x0_ref.at[d0], sem).start()
            pltpu.make_async_copy(out_ref.at[r], x1_ref.at[d1], sem).start()
        return 0

    def drain(r, _):
        pltpu.make_async_copy(zero_ref.at[0], x0_ref.at[0], sem).wait()
        return 0

    for c in range(TA // CHUNK):
        jax.lax.fori_loop(c * CHUNK, (c + 1) * CHUNK, issue, 0)
        jax.lax.fori_loop(0, 2 * CHUNK, drain, 0)


def kernel(x, W):
    out, x0, x1 = pl.pallas_call(
        _router_kernel,
        grid=(NB,),
        in_specs=[
            pl.BlockSpec((TA, D), lambda i: (i, 0)),
            pl.BlockSpec((D, E), lambda i: (0, 0)),
        ],
        out_specs=[
            pl.BlockSpec((TA, D), lambda i: (i, 0)),
            pl.BlockSpec(memory_space=pltpu.MemorySpace.HBM),
            pl.BlockSpec(memory_space=pltpu.MemorySpace.HBM),
        ],
        out_shape=[
            jax.ShapeDtypeStruct((N, D), jnp.float32),
            jax.ShapeDtypeStruct((N, D), jnp.float32),
            jax.ShapeDtypeStruct((N, D), jnp.float32),
        ],
        scratch_shapes=[
            pltpu.SMEM((2,), jnp.int32),
            pltpu.VMEM((1, TA), jnp.int32),
            pltpu.VMEM((1, TA), jnp.int32),
            pltpu.VMEM((1, TA), jnp.int32),
            pltpu.SMEM((1, TA), jnp.int32),
            pltpu.SMEM((1, TA), jnp.int32),
            pltpu.SMEM((1, TA), jnp.int32),
            pltpu.VMEM((1, D), jnp.float32),
            pltpu.SemaphoreType.DMA,
            pltpu.SemaphoreType.DMA,
        ],
        compiler_params=pltpu.CompilerParams(
            dimension_semantics=("arbitrary",)),
    )(x, W)

    return (x0, x1, out)


# speedup vs baseline: 3.5330x; 1.2804x over previous
import jax
import jax.numpy as jnp
from jax.experimental import pallas as pl
from jax.experimental.pallas import tpu as pltpu

N, D, E = 32768, 768, 2
TA = 1024
NB = N // TA
CHUNK = 256
ZR = 512


def _router_kernel(x_ref, w_ref, out_ref, x0_ref, x1_ref,
                   cnt, d0v, d1v, sv, d0s, d1s, ss, zero_ref, zbuf,
                   isem, sem, zsem):
    @pl.when(pl.program_id(0) == 0)
    def _():
        cnt[0] = 0
        cnt[1] = 0
        zero_ref[...] = jnp.zeros((1, D), jnp.float32)
        zbuf[...] = jnp.zeros((ZR, D), jnp.float32)

    x = x_ref[...]
    logits = jax.lax.dot_general(x, w_ref[...], (((1,), (0,)), ((), ())),
                                 preferred_element_type=jnp.float32)
    lt = jnp.transpose(logits)
    l0 = lt[0:1, :]
    l1 = lt[1:2, :]
    m = jnp.maximum(l0, l1)
    e0 = jnp.exp(l0 - m)
    e1 = jnp.exp(l1 - m)
    den = e0 + e1
    g0 = e0 / den
    g1 = e1 / den
    selb = g1 > g0
    val = jnp.maximum(g0, g1)
    out_ref[...] = x * jnp.transpose(val)

    ii = jax.lax.broadcasted_iota(jnp.int32, (TA, TA), 0)
    jj = jax.lax.broadcasted_iota(jnp.int32, (TA, TA), 1)
    tri = (ii < jj).astype(jnp.float32)
    m1 = selb.astype(jnp.float32)
    pos1 = jax.lax.dot_general(m1, tri, (((1,), (0,)), ((), ())),
                               preferred_element_type=jnp.float32)
    ar = jax.lax.broadcasted_iota(jnp.int32, (1, TA), 1).astype(jnp.float32)
    pos0 = ar - pos1

    c0 = cnt[0]
    c1 = cnt[1]
    dest0 = c0 + pos0.astype(jnp.int32)
    dest1 = c1 + pos1.astype(jnp.int32)
    d0v[...] = jnp.where(selb, N - 1 - dest1, dest0)
    d1v[...] = jnp.where(selb, dest1, N - 1 - dest0)
    sv[...] = selb.astype(jnp.int32)

    t1 = jnp.sum(m1).astype(jnp.int32)
    cnt[0] = c0 + (TA - t1)
    cnt[1] = c1 + t1

    cp0 = pltpu.make_async_copy(d0v, d0s, isem)
    cp1 = pltpu.make_async_copy(d1v, d1s, isem)
    cp2 = pltpu.make_async_copy(sv, ss, isem)
    cp0.start(); cp1.start(); cp2.start()
    cp0.wait(); cp1.wait(); cp2.wait()

    def issue(r, _):
        d0 = d0s[0, r]
        d1 = d1s[0, r]
        s = ss[0, r]

        @pl.when(s == 0)
        def _():
            pltpu.make_async_copy(out_ref.at[r], x0_ref.at[d0], sem).start()

        @pl.when(s == 1)
        def _():
            pltpu.make_async_copy(out_ref.at[r], x1_ref.at[d1], sem).start()
        return 0

    def drain(r, _):
        pltpu.make_async_copy(zero_ref.at[0], x0_ref.at[0], sem).wait()
        return 0

    for c in range(TA // CHUNK):
        jax.lax.fori_loop(c * CHUNK, (c + 1) * CHUNK, issue, 0)
        jax.lax.fori_loop(0, CHUNK, drain, 0)

    @pl.when(pl.program_id(0) == NB - 1)
    def _():
        c0f = cnt[0]
        c1f = cnt[1]
        h0 = (-c0f) % 8
        h1 = (-c1f) % 8

        def hfill0(i, _):
            pltpu.make_async_copy(zero_ref.at[0], x0_ref.at[c0f + i],
                                  sem).start()
            return 0

        def hfill1(i, _):
            pltpu.make_async_copy(zero_ref.at[0], x1_ref.at[c1f + i],
                                  sem).start()
            return 0

        def hdrain(i, _):
            pltpu.make_async_copy(zero_ref.at[0], x0_ref.at[0], sem).wait()
            return 0

        jax.lax.fori_loop(0, h0, hfill0, 0)
        jax.lax.fori_loop(0, h1, hfill1, 0)
        jax.lax.fori_loop(0, h0 + h1, hdrain, 0)

        a0 = c0f + h0
        a1 = c1f + h1
        k0 = (N - a0 + ZR - 1) // ZR
        k1 = (N - a1 + ZR - 1) // ZR

        def zfill0(i, _):
            off = pl.multiple_of(jnp.minimum(a0 + i * ZR, N - ZR), 8)
            pltpu.make_async_copy(zbuf, x0_ref.at[pl.ds(off, ZR)],
                                  zsem).start()
            return 0

        def zfill1(i, _):
            off = pl.multiple_of(jnp.minimum(a1 + i * ZR, N - ZR), 8)
            pltpu.make_async_copy(zbuf, x1_ref.at[pl.ds(off, ZR)],
                                  zsem).start()
            return 0

        def zdrain(i, _):
            pltpu.make_async_copy(zbuf, x0_ref.at[pl.ds(0, ZR)],
                                  zsem).wait()
            return 0

        jax.lax.fori_loop(0, k0, zfill0, 0)
        jax.lax.fori_loop(0, k1, zfill1, 0)
        jax.lax.fori_loop(0, k0 + k1, zdrain, 0)


def kernel(x, W):
    out, x0, x1 = pl.pallas_call(
        _router_kernel,
        grid=(NB,),
        in_specs=[
            pl.BlockSpec((TA, D), lambda i: (i, 0)),
            pl.BlockSpec((D, E), lambda i: (0, 0)),
        ],
        out_specs=[
            pl.BlockSpec((TA, D), lambda i: (i, 0)),
            pl.BlockSpec(memory_space=pltpu.MemorySpace.HBM),
            pl.BlockSpec(memory_space=pltpu.MemorySpace.HBM),
        ],
        out_shape=[
            jax.ShapeDtypeStruct((N, D), jnp.float32),
            jax.ShapeDtypeStruct((N, D), jnp.float32),
            jax.ShapeDtypeStruct((N, D), jnp.float32),
        ],
        scratch_shapes=[
            pltpu.SMEM((2,), jnp.int32),
            pltpu.VMEM((1, TA), jnp.int32),
            pltpu.VMEM((1, TA), jnp.int32),
            pltpu.VMEM((1, TA), jnp.int32),
            pltpu.SMEM((1, TA), jnp.int32),
            pltpu.SMEM((1, TA), jnp.int32),
            pltpu.SMEM((1, TA), jnp.int32),
            pltpu.VMEM((1, D), jnp.float32),
            pltpu.VMEM((ZR, D), jnp.float32),
            pltpu.SemaphoreType.DMA,
            pltpu.SemaphoreType.DMA,
            pltpu.SemaphoreType.DMA,
        ],
        compiler_params=pltpu.CompilerParams(
            dimension_semantics=("arbitrary",)),
    )(x, W)

    return (x0, x1, out)
